# baseline (speedup 1.0000x reference)
import jax
import jax.numpy as jnp
from jax.experimental import pallas as pl

_ROWS_PER_BLOCK = 1024


def _copy_block(x_ref, o_ref):
    o_ref[...] = x_ref[...]


def kernel(x, idx, label):
    del idx, label
    rows, cols = x.shape
    grid = rows // _ROWS_PER_BLOCK
    return pl.pallas_call(
        _copy_block,
        out_shape=jax.ShapeDtypeStruct(x.shape, x.dtype),
        grid=(grid,),
        in_specs=[pl.BlockSpec((_ROWS_PER_BLOCK, cols), lambda i: (i, 0))],
        out_specs=pl.BlockSpec((_ROWS_PER_BLOCK, cols), lambda i: (i, 0)),
    )(x)


# speedup vs baseline: 1.2652x; 1.2652x over previous
import jax
import jax.numpy as jnp
from jax.experimental import pallas as pl
from jax.experimental.pallas import tpu as pltpu

_NUM_CHUNKS = 4
_ROWS = 1024
_COLS = 1024
_CHUNK_ROWS = _ROWS // _NUM_CHUNKS


def _stream_copy(x_hbm, o_hbm, *rest):
    bufs = rest[:_NUM_CHUNKS]
    in_sems = rest[_NUM_CHUNKS:2 * _NUM_CHUNKS]
    out_sems = rest[2 * _NUM_CHUNKS:]
    ins = [
        pltpu.make_async_copy(
            x_hbm.at[pl.ds(i * _CHUNK_ROWS, _CHUNK_ROWS), :], bufs[i], in_sems[i])
        for i in range(_NUM_CHUNKS)
    ]
    outs = [
        pltpu.make_async_copy(
            bufs[i], o_hbm.at[pl.ds(i * _CHUNK_ROWS, _CHUNK_ROWS), :], out_sems[i])
        for i in range(_NUM_CHUNKS)
    ]
    for c in ins:
        c.start()
    for i in range(_NUM_CHUNKS):
        ins[i].wait()
        outs[i].start()
    for c in outs:
        c.wait()


def kernel(x, idx, label):
    del idx, label
    return pl.pallas_call(
        _stream_copy,
        out_shape=jax.ShapeDtypeStruct(x.shape, x.dtype),
        in_specs=[pl.BlockSpec(memory_space=pl.ANY)],
        out_specs=pl.BlockSpec(memory_space=pl.ANY),
        scratch_shapes=(
            [pltpu.VMEM((_CHUNK_ROWS, _COLS), jnp.float32)] * _NUM_CHUNKS
            + [pltpu.SemaphoreType.DMA] * (2 * _NUM_CHUNKS)
        ),
    )(x)
